# Initial kernel scaffold; baseline (speedup 1.0000x reference)
import jax
import jax.numpy as jnp
from jax.experimental import pallas as pl


def kernel(x, edge_index, emb_table, W1, b1, W2, b2):
    raise NotImplementedError("write your pallas kernel here")


# speedup vs baseline: 8.4006x; 8.4006x over previous
import functools
import jax
import jax.numpy as jnp
from jax import lax
from jax.experimental import pallas as pl
from jax.experimental.pallas import tpu as pltpu
from jax.experimental.pallas import tpu_sc as plsc

NC = 2
NS = 16
NW = NC * NS

N_NODES = 10000
D = 128
R = 10240
DUMMY = N_NODES
E = 320000
EPT_REAL = E // NW
CHUNK = 128
CPT = 80
EPT = CHUNK * CPT
ROWS_PT = R // NW
ROWS_PSC = R // NS


def _mesh():
    return plsc.VectorSubcoreMesh(
        core_axis_name="c", subcore_axis_name="s", num_cores=NC, num_subcores=NS
    )


def _sc_emb_hist(xpad, emb, dstf, z1):
    kfn = pl.kernel(
        functools.partial(_sc_emb_hist_body),
        out_type=(
            jax.ShapeDtypeStruct((R, D), jnp.float32),
            jax.ShapeDtypeStruct((NC, R), jnp.float32),
        ),
        mesh=_mesh(),
        scratch_types=[
            pltpu.VMEM((CHUNK,), jnp.int32),
            pltpu.VMEM((64,), jnp.int32),
            pltpu.VMEM((CHUNK, D), jnp.float32),
            pltpu.VMEM((64, D), jnp.float32),
            pltpu.VMEM((CHUNK,), jnp.int32),
            pltpu.VMEM((CHUNK,), jnp.float32),
            pltpu.VMEM_SHARED((R,), jnp.float32),
            pltpu.SemaphoreType.DMA,
        ],
    )
    return kfn(xpad, emb, dstf, z1)


def _sc_emb_hist_body(x_hbm, emb_hbm, dst_hbm, z1_hbm, h_out, hist_out,
                      xb_a, xb_b, rb_a, rb_b, didx, ones_v, hist_sp, sem):
    c = lax.axis_index("c")
    s = lax.axis_index("s")
    wid = c * NS + s

    rowbase = wid * ROWS_PT
    off = 0
    for xb, rb, size in ((xb_a, rb_a, CHUNK), (xb_a, rb_a, CHUNK), (xb_b, rb_b, 64)):
        pltpu.sync_copy(x_hbm.at[pl.ds(rowbase + off, size)], xb)
        pltpu.async_copy(emb_hbm.at[xb], rb, sem).wait()
        pltpu.sync_copy(rb, h_out.at[pl.ds(rowbase + off, size)])
        off += size

    for j in range(CHUNK // 16):
        ones_v[pl.ds(j * 16, 16)] = jnp.ones((16,), jnp.float32)

    pltpu.sync_copy(z1_hbm, hist_sp.at[pl.ds(s * ROWS_PSC, ROWS_PSC)])
    plsc.subcore_barrier()

    ebase = wid * EPT

    def hbody(k, carry):
        pltpu.sync_copy(dst_hbm.at[pl.ds(ebase + k * CHUNK, CHUNK)], didx)
        pltpu.sync_copy(ones_v, hist_sp.at[didx], add=True)
        return carry

    lax.fori_loop(0, CPT, hbody, 0)
    plsc.subcore_barrier()

    pltpu.sync_copy(hist_sp.at[pl.ds(s * ROWS_PSC, ROWS_PSC)],
                    hist_out.at[c, pl.ds(s * ROWS_PSC, ROWS_PSC)])


def _sc_edge_scatter(g, srcf, dstf, z2):
    kfn = pl.kernel(
        functools.partial(_sc_edge_body),
        out_type=jax.ShapeDtypeStruct((NC, R, D), jnp.float32),
        mesh=_mesh(),
        scratch_types=[
            pltpu.VMEM((CHUNK,), jnp.int32),
            pltpu.VMEM((CHUNK,), jnp.int32),
            pltpu.VMEM((CHUNK, D), jnp.float32),
            pltpu.VMEM_SHARED((R, D), jnp.float32),
            pltpu.SemaphoreType.DMA,
        ],
    )
    return kfn(g, srcf, dstf, z2)


def _sc_edge_body(g_hbm, src_hbm, dst_hbm, z2_hbm, part_out,
                  sidx, didx, rows, S_sp, sem):
    c = lax.axis_index("c")
    s = lax.axis_index("s")
    wid = c * NS + s

    pltpu.sync_copy(z2_hbm, S_sp.at[pl.ds(s * ROWS_PSC, ROWS_PSC)])
    plsc.subcore_barrier()

    ebase = wid * EPT

    def ebody(k, carry):
        pltpu.sync_copy(src_hbm.at[pl.ds(ebase + k * CHUNK, CHUNK)], sidx)
        pltpu.sync_copy(dst_hbm.at[pl.ds(ebase + k * CHUNK, CHUNK)], didx)
        pltpu.async_copy(g_hbm.at[sidx], rows, sem).wait()
        pltpu.sync_copy(rows, S_sp.at[didx], add=True)
        return carry

    lax.fori_loop(0, CPT, ebody, 0)
    plsc.subcore_barrier()

    pltpu.sync_copy(S_sp.at[pl.ds(s * ROWS_PSC, ROWS_PSC)],
                    part_out.at[c, pl.ds(s * ROWS_PSC, ROWS_PSC)])


def _tc1_body(deg_ref, h_ref, w_ref, g_ref, dis_ref):
    dis = lax.rsqrt(deg_ref[...])
    hw = jnp.dot(h_ref[...], w_ref[...], preferred_element_type=jnp.float32)
    g_ref[...] = dis * hw
    dis_ref[...] = dis


def _tc1(deg_col, h, W1):
    return pl.pallas_call(
        _tc1_body,
        out_shape=(
            jax.ShapeDtypeStruct((R, D), jnp.float32),
            jax.ShapeDtypeStruct((R, 1), jnp.float32),
        ),
    )(deg_col, h, W1)


def _tc2_body(sp_ref, g_ref, dis_ref, b_ref, w_ref, out_ref):
    dis = dis_ref[...]
    acc = sp_ref[0] + sp_ref[1] + g_ref[...]
    h2 = jnp.maximum(dis * acc + b_ref[...], 0.0)
    hw = jnp.dot(h2, w_ref[...], preferred_element_type=jnp.float32)
    out_ref[...] = dis * hw


def _tc2(S1, g1, dis, b1, W2):
    return pl.pallas_call(
        _tc2_body,
        out_shape=jax.ShapeDtypeStruct((R, D), jnp.float32),
    )(S1, g1, dis, b1, W2)


def _tc3_body(sp_ref, g_ref, dis_ref, b_ref, out_ref):
    acc = sp_ref[0] + sp_ref[1] + g_ref[...]
    out_ref[...] = dis_ref[...] * acc + b_ref[...]


def _tc3(S2, g2, dis, b2):
    return pl.pallas_call(
        _tc3_body,
        out_shape=jax.ShapeDtypeStruct((R, D), jnp.float32),
    )(S2, g2, dis, b2)


@jax.jit
def _impl(x, edge_index, emb_table, W1, b1, W2, b2):
    x32 = x.astype(jnp.int32)
    xpad = jnp.concatenate([x32, jnp.zeros((R - N_NODES,), jnp.int32)])

    src = edge_index[0].astype(jnp.int32)
    dst = edge_index[1].astype(jnp.int32)
    pad = EPT - EPT_REAL
    srcf = jnp.concatenate(
        [src.reshape(NW, EPT_REAL), jnp.zeros((NW, pad), jnp.int32)], axis=1
    ).reshape(-1)
    dstf = jnp.concatenate(
        [dst.reshape(NW, EPT_REAL), jnp.full((NW, pad), DUMMY, jnp.int32)], axis=1
    ).reshape(-1)

    z1 = jnp.zeros((ROWS_PSC,), jnp.float32)
    z2 = jnp.zeros((ROWS_PSC, D), jnp.float32)

    h, hist = _sc_emb_hist(xpad, emb_table, dstf, z1)
    deg_col = (hist[0] + hist[1] + 1.0).reshape(R, 1)

    g1, dis = _tc1(deg_col, h, W1)
    S1 = _sc_edge_scatter(g1, srcf, dstf, z2)
    g2 = _tc2(S1, g1, dis, b1.reshape(1, D), W2)
    S2 = _sc_edge_scatter(g2, srcf, dstf, z2)
    out = _tc3(S2, g2, dis, b2.reshape(1, D))
    return out[:N_NODES]


def kernel(x, edge_index, emb_table, W1, b1, W2, b2):
    return _impl(x, edge_index, emb_table, W1, b1, W2, b2)


# speedup vs baseline: 11.0822x; 1.3192x over previous
import functools
import jax
import jax.numpy as jnp
from jax import lax
from jax.experimental import pallas as pl
from jax.experimental.pallas import tpu as pltpu
from jax.experimental.pallas import tpu_sc as plsc

NC = 2
NS = 16
NW = NC * NS

N_NODES = 10000
D = 128
R = 10240
DUMMY = N_NODES
E = 320000
EPT_REAL = E // NW
CHUNK = 128
CPT = 80
EPT = CHUNK * CPT
ROWS_PT = R // NW
ROWS_PSC = R // NS


def _mesh():
    return plsc.VectorSubcoreMesh(
        core_axis_name="c", subcore_axis_name="s", num_cores=NC, num_subcores=NS
    )


def _sc_emb_hist(xpad, emb, dstf, z1):
    kfn = pl.kernel(
        functools.partial(_sc_emb_hist_body),
        out_type=(
            jax.ShapeDtypeStruct((R, D), jnp.float32),
            jax.ShapeDtypeStruct((NC, R), jnp.float32),
        ),
        mesh=_mesh(),
        scratch_types=[
            pltpu.VMEM((CHUNK,), jnp.int32),
            pltpu.VMEM((64,), jnp.int32),
            pltpu.VMEM((CHUNK, D), jnp.float32),
            pltpu.VMEM((64, D), jnp.float32),
            pltpu.VMEM((CPT, CHUNK), jnp.int32),
            pltpu.VMEM((CHUNK,), jnp.float32),
            pltpu.VMEM_SHARED((R,), jnp.float32),
            pltpu.SemaphoreType.DMA,
        ],
    )
    return kfn(xpad, emb, dstf, z1)


def _sc_emb_hist_body(x_hbm, emb_hbm, dst_hbm, z1_hbm, h_out, hist_out,
                      xb_a, xb_b, rb_a, rb_b, didx_all, ones_v, hist_sp, sem):
    c = lax.axis_index("c")
    s = lax.axis_index("s")
    wid = c * NS + s

    rowbase = wid * ROWS_PT
    off = 0
    for xb, rb, size in ((xb_a, rb_a, CHUNK), (xb_a, rb_a, CHUNK), (xb_b, rb_b, 64)):
        pltpu.sync_copy(x_hbm.at[pl.ds(rowbase + off, size)], xb)
        pltpu.async_copy(emb_hbm.at[xb], rb, sem).wait()
        pltpu.sync_copy(rb, h_out.at[pl.ds(rowbase + off, size)])
        off += size

    for j in range(CHUNK // 16):
        ones_v[pl.ds(j * 16, 16)] = jnp.ones((16,), jnp.float32)

    pltpu.sync_copy(z1_hbm, hist_sp.at[pl.ds(s * ROWS_PSC, ROWS_PSC)])
    pltpu.sync_copy(dst_hbm.at[wid], didx_all)
    plsc.subcore_barrier()

    def hbody(k, carry):
        pltpu.sync_copy(ones_v, hist_sp.at[didx_all.at[k]], add=True)
        return carry

    lax.fori_loop(0, CPT, hbody, 0)
    plsc.subcore_barrier()

    pltpu.sync_copy(hist_sp.at[pl.ds(s * ROWS_PSC, ROWS_PSC)],
                    hist_out.at[c, pl.ds(s * ROWS_PSC, ROWS_PSC)])


NBUF = 2
HCPT = CPT // 2


def _sc_edge_scatter(g, srcf, dstf, z2):
    kfn = pl.kernel(
        functools.partial(_sc_edge_body),
        out_type=jax.ShapeDtypeStruct((NC, R, D), jnp.float32),
        mesh=_mesh(),
        scratch_types=[
            pltpu.VMEM((HCPT, CHUNK), jnp.int32),
            pltpu.VMEM((HCPT, CHUNK), jnp.int32),
            [pltpu.VMEM((CHUNK, D), jnp.float32) for _ in range(NBUF)],
            pltpu.VMEM_SHARED((R, D), jnp.float32),
            [pltpu.SemaphoreType.DMA for _ in range(NBUF)],
        ],
    )
    return kfn(g, srcf, dstf, z2)


def _sc_edge_body(g_hbm, src_hbm, dst_hbm, z2_hbm, part_out,
                  sidx_all, didx_all, rows, S_sp, sems):
    c = lax.axis_index("c")
    s = lax.axis_index("s")
    wid = c * NS + s

    pltpu.sync_copy(z2_hbm, S_sp.at[pl.ds(s * ROWS_PSC, ROWS_PSC)])
    plsc.subcore_barrier()

    for p in range(CPT // HCPT):
        pltpu.sync_copy(src_hbm.at[wid, pl.ds(p * HCPT, HCPT)], sidx_all)
        pltpu.sync_copy(dst_hbm.at[wid, pl.ds(p * HCPT, HCPT)], didx_all)

        for b in range(NBUF):
            pltpu.async_copy(g_hbm.at[sidx_all.at[b]], rows[b], sems[b])

        def outer(i, carry):
            for b in range(NBUF):
                k = i * NBUF + b
                pltpu.make_async_copy(
                    g_hbm.at[sidx_all.at[k]], rows[b], sems[b]).wait()
                pltpu.sync_copy(rows[b], S_sp.at[didx_all.at[k]], add=True)
                pltpu.async_copy(
                    g_hbm.at[sidx_all.at[k + NBUF]], rows[b], sems[b])
            return carry

        lax.fori_loop(0, HCPT // NBUF - 1, outer, 0)
        for b in range(NBUF):
            k = HCPT - NBUF + b
            pltpu.make_async_copy(
                g_hbm.at[sidx_all.at[k]], rows[b], sems[b]).wait()
            pltpu.sync_copy(rows[b], S_sp.at[didx_all.at[k]], add=True)
    plsc.subcore_barrier()

    pltpu.sync_copy(S_sp.at[pl.ds(s * ROWS_PSC, ROWS_PSC)],
                    part_out.at[c, pl.ds(s * ROWS_PSC, ROWS_PSC)])


def _tc1_body(deg_ref, h_ref, w_ref, g_ref, dis_ref):
    dis = lax.rsqrt(deg_ref[...])
    hw = jnp.dot(h_ref[...], w_ref[...], preferred_element_type=jnp.float32)
    g_ref[...] = dis * hw
    dis_ref[...] = dis


def _tc1(deg_col, h, W1):
    return pl.pallas_call(
        _tc1_body,
        out_shape=(
            jax.ShapeDtypeStruct((R, D), jnp.float32),
            jax.ShapeDtypeStruct((R, 1), jnp.float32),
        ),
    )(deg_col, h, W1)


def _tc2_body(sp_ref, g_ref, dis_ref, b_ref, w_ref, out_ref):
    dis = dis_ref[...]
    acc = sp_ref[0] + sp_ref[1] + g_ref[...]
    h2 = jnp.maximum(dis * acc + b_ref[...], 0.0)
    hw = jnp.dot(h2, w_ref[...], preferred_element_type=jnp.float32)
    out_ref[...] = dis * hw


def _tc2(S1, g1, dis, b1, W2):
    return pl.pallas_call(
        _tc2_body,
        out_shape=jax.ShapeDtypeStruct((R, D), jnp.float32),
    )(S1, g1, dis, b1, W2)


def _tc3_body(sp_ref, g_ref, dis_ref, b_ref, out_ref):
    acc = sp_ref[0] + sp_ref[1] + g_ref[...]
    out_ref[...] = dis_ref[...] * acc + b_ref[...]


def _tc3(S2, g2, dis, b2):
    return pl.pallas_call(
        _tc3_body,
        out_shape=jax.ShapeDtypeStruct((R, D), jnp.float32),
    )(S2, g2, dis, b2)


@jax.jit
def _impl(x, edge_index, emb_table, W1, b1, W2, b2):
    x32 = x.astype(jnp.int32)
    xpad = jnp.concatenate([x32, jnp.zeros((R - N_NODES,), jnp.int32)])

    src = edge_index[0].astype(jnp.int32)
    dst = edge_index[1].astype(jnp.int32)
    pad = EPT - EPT_REAL
    srcf = jnp.concatenate(
        [src.reshape(NW, EPT_REAL), jnp.zeros((NW, pad), jnp.int32)], axis=1
    ).reshape(NW, CPT, CHUNK)
    dstf = jnp.concatenate(
        [dst.reshape(NW, EPT_REAL), jnp.full((NW, pad), DUMMY, jnp.int32)], axis=1
    ).reshape(NW, CPT, CHUNK)

    z1 = jnp.zeros((ROWS_PSC,), jnp.float32)
    z2 = jnp.zeros((ROWS_PSC, D), jnp.float32)

    h, hist = _sc_emb_hist(xpad, emb_table, dstf, z1)
    deg_col = (hist[0] + hist[1] + 1.0).reshape(R, 1)

    g1, dis = _tc1(deg_col, h, W1)
    S1 = _sc_edge_scatter(g1, srcf, dstf, z2)
    g2 = _tc2(S1, g1, dis, b1.reshape(1, D), W2)
    S2 = _sc_edge_scatter(g2, srcf, dstf, z2)
    out = _tc3(S2, g2, dis, b2.reshape(1, D))
    return out[:N_NODES]


def kernel(x, edge_index, emb_table, W1, b1, W2, b2):
    return _impl(x, edge_index, emb_table, W1, b1, W2, b2)
